# baseline (speedup 1.0000x reference)
import functools
import math

import jax
import jax.numpy as jnp
from jax.experimental import pallas as pl
from jax.experimental.pallas import tpu as pltpu


def _loss_body(fp_ref, tg_ref, pv_ref, mcao_ref, out_ref, acc_ref, *,
               inv_n_final, inv_n_mcao, alpha, beta, gamma, delta,
               bce_pos, bce_neg):
    step = pl.program_id(0)
    nsteps = pl.num_programs(0)

    @pl.when(step == 0)
    def _init():
        acc_ref[...] = jnp.zeros_like(acc_ref)

    x = mcao_ref[...]
    acc_ref[...] += jnp.sum(jnp.abs(x), axis=(0, 1))[None, :]

    @pl.when(step == nsteps - 1)
    def _finalize():
        fp = fp_ref[...]
        tg = tg_ref[...]
        pv = pv_ref[...]

        diff = fp - tg
        pred_diff = fp - pv
        target_diff = tg - pv

        label = jnp.where(target_diff > 0.0, 1.0, 0.0)
        bce = jnp.where(pred_diff > 0.0, bce_pos - label, bce_neg)

        stacked = jnp.concatenate([diff * diff, bce, jnp.abs(pred_diff)],
                                  axis=0)
        part = jnp.sum(stacked, axis=1, keepdims=True)

        mse = part[0, 0] * inv_n_final
        direction = part[1, 0] * inv_n_final
        smoothness = part[2, 0] * inv_n_final
        mcao_reg = jnp.sum(acc_ref[...]) * inv_n_mcao

        out_ref[0] = (alpha * mse + beta * direction
                      + gamma * smoothness + delta * mcao_reg)
        out_ref[1] = mse
        out_ref[2] = direction
        out_ref[3] = smoothness
        out_ref[4] = mcao_reg


def _batch_block(B, S, C):
    target = max(1, (4 * 1024 * 1024) // (S * C * 4))
    bb = 1
    for cand in range(1, B + 1):
        if B % cand == 0 and cand <= target:
            bb = cand
    return bb


def kernel(predictions, targets, prev_price, mcao_features):
    B, S, P = predictions.shape
    n_final = B * P

    final_pred = jax.lax.slice_in_dim(predictions, S - 1, S, axis=1)
    final_pred = final_pred.reshape(1, n_final).astype(jnp.float32)
    targets2d = targets.reshape(1, n_final).astype(jnp.float32)
    prev2d = jnp.broadcast_to(prev_price.reshape(B, 1).astype(jnp.float32),
                              (B, P)).reshape(1, n_final)

    MB, MS, MC = mcao_features.shape
    n_mcao = MB * MS * MC
    mcao = mcao_features.astype(jnp.float32)
    bb = _batch_block(MB, MS, MC)
    nsteps = MB // bb

    body = functools.partial(
        _loss_body,
        inv_n_final=1.0 / float(n_final),
        inv_n_mcao=1.0 / float(n_mcao),
        alpha=0.6, beta=0.3, gamma=0.05, delta=0.05,
        bce_pos=1.0 + math.log1p(math.exp(-1.0)),
        bce_neg=math.log(2.0))

    out = pl.pallas_call(
        body,
        out_shape=jax.ShapeDtypeStruct((5,), jnp.float32),
        grid=(nsteps,),
        in_specs=[
            pl.BlockSpec((1, n_final), lambda i: (0, 0)),
            pl.BlockSpec((1, n_final), lambda i: (0, 0)),
            pl.BlockSpec((1, n_final), lambda i: (0, 0)),
            pl.BlockSpec((bb, MS, MC), lambda i: (i, 0, 0)),
        ],
        out_specs=pl.BlockSpec(memory_space=pltpu.MemorySpace.SMEM),
        scratch_shapes=[pltpu.VMEM((1, MC), jnp.float32)],
        compiler_params=pltpu.CompilerParams(
            dimension_semantics=("arbitrary",),
            vmem_limit_bytes=48 * 1024 * 1024),
    )(final_pred, targets2d, prev2d, mcao)

    total_loss = out[0]
    components = {
        "mse": out[1],
        "direction": out[2],
        "smoothness": out[3],
        "mcao_reg": out[4],
    }
    return total_loss, components


# speedup vs baseline: 1.1864x; 1.1864x over previous
import functools
import math

import jax
import jax.numpy as jnp
from jax.experimental import pallas as pl
from jax.experimental.pallas import tpu as pltpu

_NBUF = 6


def _loss_body(fp_ref, tg_ref, pv_ref, mcao_hbm, out_ref, bufs_ref, sems,
               *, nchunks, chunk_b, inv_n_final, inv_n_mcao, alpha, beta,
               gamma, delta, bce_pos, bce_neg):
    C = bufs_ref.shape[-1]

    def _copy(chunk, slot):
        return pltpu.make_async_copy(
            mcao_hbm.at[pl.ds(chunk * chunk_b, chunk_b)],
            bufs_ref.at[slot],
            sems.at[slot])

    for i in range(min(_NBUF, nchunks)):
        _copy(i, i).start()

    def _loop(i, acc):
        slot = jax.lax.rem(i, _NBUF)
        _copy(i, slot).wait()
        x = bufs_ref[slot]

        @pl.when(i + _NBUF < nchunks)
        def _():
            _copy(i + _NBUF, jax.lax.rem(i + _NBUF, _NBUF)).start()

        return acc + jnp.sum(jnp.abs(x), axis=(0, 1))[None, :]

    acc = jax.lax.fori_loop(
        0, nchunks, _loop, jnp.zeros((1, C), jnp.float32))

    fp = fp_ref[...]
    tg = tg_ref[...]
    pv = pv_ref[...]

    diff = fp - tg
    pred_diff = fp - pv
    target_diff = tg - pv

    label = jnp.where(target_diff > 0.0, 1.0, 0.0)
    bce = jnp.where(pred_diff > 0.0, bce_pos - label, bce_neg)

    stacked = jnp.concatenate([diff * diff, bce, jnp.abs(pred_diff)],
                              axis=0)
    part = jnp.sum(stacked, axis=1, keepdims=True)

    mse = part[0, 0] * inv_n_final
    direction = part[1, 0] * inv_n_final
    smoothness = part[2, 0] * inv_n_final
    mcao_reg = jnp.sum(acc) * inv_n_mcao

    out_ref[0] = (alpha * mse + beta * direction
                  + gamma * smoothness + delta * mcao_reg)
    out_ref[1] = mse
    out_ref[2] = direction
    out_ref[3] = smoothness
    out_ref[4] = mcao_reg


def _chunk_batch(B, S, C):
    target = max(1, (2 * 1024 * 1024) // (S * C * 4))
    cb = 1
    for cand in range(1, B + 1):
        if B % cand == 0 and cand <= target:
            cb = cand
    return cb


def kernel(predictions, targets, prev_price, mcao_features):
    B, S, P = predictions.shape
    n_final = B * P

    final_pred = jax.lax.slice_in_dim(predictions, S - 1, S, axis=1)
    final_pred = final_pred.reshape(1, n_final).astype(jnp.float32)
    targets2d = targets.reshape(1, n_final).astype(jnp.float32)
    prev2d = jnp.broadcast_to(prev_price.reshape(B, 1).astype(jnp.float32),
                              (B, P)).reshape(1, n_final)

    MB, MS, MC = mcao_features.shape
    n_mcao = MB * MS * MC
    mcao = mcao_features.astype(jnp.float32)
    chunk_b = _chunk_batch(MB, MS, MC)
    nchunks = MB // chunk_b

    body = functools.partial(
        _loss_body,
        nchunks=nchunks,
        chunk_b=chunk_b,
        inv_n_final=1.0 / float(n_final),
        inv_n_mcao=1.0 / float(n_mcao),
        alpha=0.6, beta=0.3, gamma=0.05, delta=0.05,
        bce_pos=1.0 + math.log1p(math.exp(-1.0)),
        bce_neg=math.log(2.0))

    out = pl.pallas_call(
        body,
        out_shape=jax.ShapeDtypeStruct((5,), jnp.float32),
        in_specs=[
            pl.BlockSpec((1, n_final), lambda: (0, 0)),
            pl.BlockSpec((1, n_final), lambda: (0, 0)),
            pl.BlockSpec((1, n_final), lambda: (0, 0)),
            pl.BlockSpec(memory_space=pltpu.MemorySpace.HBM),
        ],
        out_specs=pl.BlockSpec(memory_space=pltpu.MemorySpace.SMEM),
        scratch_shapes=[
            pltpu.VMEM((_NBUF, chunk_b, MS, MC), jnp.float32),
            pltpu.SemaphoreType.DMA((_NBUF,)),
        ],
        compiler_params=pltpu.CompilerParams(
            vmem_limit_bytes=48 * 1024 * 1024),
    )(final_pred, targets2d, prev2d, mcao)

    total_loss = out[0]
    components = {
        "mse": out[1],
        "direction": out[2],
        "smoothness": out[3],
        "mcao_reg": out[4],
    }
    return total_loss, components


# speedup vs baseline: 1.2039x; 1.0148x over previous
import functools
import math

import jax
import jax.numpy as jnp
from jax.experimental import pallas as pl
from jax.experimental.pallas import tpu as pltpu

_NBUF = 6


def _loss_body(fp_ref, tg_ref, pv_ref, mcao_hbm, out_ref, bufs_ref, sems,
               *, nchunks, chunk_b, inv_n_final, inv_n_mcao, alpha, beta,
               gamma, delta, bce_pos, bce_neg):
    C = bufs_ref.shape[-1]

    def _copy(chunk, slot):
        return pltpu.make_async_copy(
            mcao_hbm.at[pl.ds(chunk * chunk_b, chunk_b)],
            bufs_ref.at[slot],
            sems.at[slot])

    for i in range(min(_NBUF, nchunks)):
        _copy(i, i).start()

    fp = fp_ref[...]
    tg = tg_ref[...]
    pv = pv_ref[...]

    diff = fp - tg
    pred_diff = fp - pv
    target_diff = tg - pv

    label = jnp.where(target_diff > 0.0, 1.0, 0.0)
    bce = jnp.where(pred_diff > 0.0, bce_pos - label, bce_neg)

    stacked = jnp.concatenate([diff * diff, bce, jnp.abs(pred_diff)],
                              axis=0)
    part = jnp.sum(stacked, axis=1, keepdims=True)

    def _loop(i, acc):
        slot = jax.lax.rem(i, _NBUF)
        _copy(i, slot).wait()
        x = bufs_ref[slot]

        @pl.when(i + _NBUF < nchunks)
        def _():
            _copy(i + _NBUF, jax.lax.rem(i + _NBUF, _NBUF)).start()

        return acc + jnp.sum(jnp.abs(x), axis=(0, 1))[None, :]

    acc = jax.lax.fori_loop(
        0, nchunks, _loop, jnp.zeros((1, C), jnp.float32))

    mse = part[0, 0] * inv_n_final
    direction = part[1, 0] * inv_n_final
    smoothness = part[2, 0] * inv_n_final
    mcao_reg = jnp.sum(acc) * inv_n_mcao

    out_ref[0] = (alpha * mse + beta * direction
                  + gamma * smoothness + delta * mcao_reg)
    out_ref[1] = mse
    out_ref[2] = direction
    out_ref[3] = smoothness
    out_ref[4] = mcao_reg


def _chunk_batch(B, S, C):
    target = max(1, (2 * 1024 * 1024) // (S * C * 4))
    cb = 1
    for cand in range(1, B + 1):
        if B % cand == 0 and cand <= target:
            cb = cand
    return cb


def kernel(predictions, targets, prev_price, mcao_features):
    B, S, P = predictions.shape
    n_final = B * P

    final_pred = jax.lax.slice_in_dim(predictions, S - 1, S, axis=1)
    final_pred = final_pred.reshape(1, n_final).astype(jnp.float32)
    targets2d = targets.reshape(1, n_final).astype(jnp.float32)
    prev2d = jnp.broadcast_to(prev_price.reshape(B, 1).astype(jnp.float32),
                              (B, P)).reshape(1, n_final)

    MB, MS, MC = mcao_features.shape
    n_mcao = MB * MS * MC
    mcao = mcao_features.astype(jnp.float32)
    chunk_b = _chunk_batch(MB, MS, MC)
    nchunks = MB // chunk_b

    body = functools.partial(
        _loss_body,
        nchunks=nchunks,
        chunk_b=chunk_b,
        inv_n_final=1.0 / float(n_final),
        inv_n_mcao=1.0 / float(n_mcao),
        alpha=0.6, beta=0.3, gamma=0.05, delta=0.05,
        bce_pos=1.0 + math.log1p(math.exp(-1.0)),
        bce_neg=math.log(2.0))

    out = pl.pallas_call(
        body,
        out_shape=jax.ShapeDtypeStruct((5,), jnp.float32),
        in_specs=[
            pl.BlockSpec((1, n_final), lambda: (0, 0)),
            pl.BlockSpec((1, n_final), lambda: (0, 0)),
            pl.BlockSpec((1, n_final), lambda: (0, 0)),
            pl.BlockSpec(memory_space=pltpu.MemorySpace.HBM),
        ],
        out_specs=pl.BlockSpec(memory_space=pltpu.MemorySpace.SMEM),
        scratch_shapes=[
            pltpu.VMEM((_NBUF, chunk_b, MS, MC), jnp.float32),
            pltpu.SemaphoreType.DMA((_NBUF,)),
        ],
        compiler_params=pltpu.CompilerParams(
            vmem_limit_bytes=48 * 1024 * 1024),
    )(final_pred, targets2d, prev2d, mcao)

    total_loss = out[0]
    components = {
        "mse": out[1],
        "direction": out[2],
        "smoothness": out[3],
        "mcao_reg": out[4],
    }
    return total_loss, components
